# Initial kernel scaffold; baseline (speedup 1.0000x reference)
import jax
import jax.numpy as jnp
from jax.experimental import pallas as pl


def kernel(token_ids, weight):
    raise NotImplementedError("write your pallas kernel here")


# speedup vs baseline: 1.6853x; 1.6853x over previous
import functools

import jax
import jax.numpy as jnp
from jax import lax
from jax.experimental import pallas as pl
from jax.experimental.pallas import tpu as pltpu
from jax.experimental.pallas import tpu_sc as plsc

NUM_EMB = 1000000
DIM = 64
TOTAL = 16384 * 50
NUM_WORKERS = 32
PER_WORKER = TOTAL // NUM_WORKERS
CHUNK = 128
NCHUNK = PER_WORKER // CHUNK


def _emb_body(idx_hbm, table_hbm, out_hbm, idx_v, rows_v, sem):
    wid = lax.axis_index("s") * 2 + lax.axis_index("c")
    base = wid * PER_WORKER

    pltpu.sync_copy(idx_hbm.at[wid], idx_v)

    def chunk_body(j, _):
        pltpu.async_copy(table_hbm.at[idx_v.at[j]], rows_v, sem).wait()
        pltpu.sync_copy(rows_v, out_hbm.at[pl.ds(base + j * CHUNK, CHUNK)])
        return 0

    lax.fori_loop(0, NCHUNK, chunk_body, 0)


@jax.jit
def _embedding_sc(token_ids, weight):
    idx = token_ids.reshape(NUM_WORKERS, NCHUNK, CHUNK)
    mesh = plsc.VectorSubcoreMesh(core_axis_name="c", subcore_axis_name="s")
    k = functools.partial(
        pl.kernel,
        mesh=mesh,
        out_type=jax.ShapeDtypeStruct((TOTAL, DIM), jnp.float32),
        scratch_types=[
            pltpu.VMEM((NCHUNK, CHUNK), jnp.int32),
            pltpu.VMEM((CHUNK, DIM), jnp.float32),
            pltpu.SemaphoreType.DMA,
        ],
        compiler_params=pltpu.CompilerParams(use_tc_tiling_on_sc=False),
    )(_emb_body)
    out = k(idx, weight)
    return out.reshape(token_ids.shape + (DIM,))


def kernel(token_ids, weight):
    return _embedding_sc(token_ids, weight)


# speedup vs baseline: 1.8731x; 1.1114x over previous
import functools

import jax
import jax.numpy as jnp
from jax import lax
from jax.experimental import pallas as pl
from jax.experimental.pallas import tpu as pltpu
from jax.experimental.pallas import tpu_sc as plsc

NUM_EMB = 1000000
DIM = 64
TOTAL = 16384 * 50
NUM_WORKERS = 32
PER_WORKER = TOTAL // NUM_WORKERS
CHUNK = 128
NCHUNK = PER_WORKER // CHUNK
GPS = 5
SUP = CHUNK * GPS
NSUP = PER_WORKER // SUP


def _emb_body(idx_hbm, table_hbm, out_hbm, idx_v, rows_a, rows_b, sga, sgb,
              soa, sob):
    wid = lax.axis_index("s") * 2 + lax.axis_index("c")
    base = wid * PER_WORKER

    pltpu.sync_copy(idx_hbm.at[wid], idx_v)

    def start_g(s, rows, sem):
        for g in range(GPS):
            pltpu.async_copy(
                table_hbm.at[idx_v.at[s * GPS + g]],
                rows.at[pl.ds(g * CHUNK, CHUNK)], sem)

    def drain_g(s, rows, sem):
        for g in range(GPS):
            pltpu.make_async_copy(
                table_hbm.at[idx_v.at[s * GPS + g]],
                rows.at[pl.ds(g * CHUNK, CHUNK)], sem).wait()

    def start_out(s, rows, sem):
        pltpu.async_copy(rows, out_hbm.at[pl.ds(base + s * SUP, SUP)], sem)

    def wait_out(s, rows, sem):
        pltpu.make_async_copy(rows, out_hbm.at[pl.ds(base + s * SUP, SUP)],
                              sem).wait()

    start_g(0, rows_a, sga)
    start_g(1, rows_b, sgb)
    drain_g(0, rows_a, sga)
    start_out(0, rows_a, soa)

    def body(i, _):
        s0 = 2 * i + 2
        s1 = 2 * i + 3
        wait_out(s0 - 2, rows_a, soa)
        start_g(s0, rows_a, sga)
        drain_g(s0 - 1, rows_b, sgb)
        start_out(s0 - 1, rows_b, sob)
        wait_out(s1 - 2, rows_b, sob)
        start_g(s1, rows_b, sgb)
        drain_g(s1 - 1, rows_a, sga)
        start_out(s1 - 1, rows_a, soa)
        return 0

    lax.fori_loop(0, (NSUP - 2) // 2, body, 0)

    drain_g(NSUP - 1, rows_b, sgb)
    start_out(NSUP - 1, rows_b, sob)
    wait_out(NSUP - 2, rows_a, soa)
    wait_out(NSUP - 1, rows_b, sob)


@jax.jit
def _embedding_sc(token_ids, weight):
    idx = token_ids.reshape(NUM_WORKERS, NCHUNK, CHUNK)
    mesh = plsc.VectorSubcoreMesh(core_axis_name="c", subcore_axis_name="s")
    k = functools.partial(
        pl.kernel,
        mesh=mesh,
        out_type=jax.ShapeDtypeStruct((TOTAL, DIM), jnp.float32),
        scratch_types=[
            pltpu.VMEM((NCHUNK, CHUNK), jnp.int32),
            pltpu.VMEM((SUP, DIM), jnp.float32),
            pltpu.VMEM((SUP, DIM), jnp.float32),
            pltpu.SemaphoreType.DMA,
            pltpu.SemaphoreType.DMA,
            pltpu.SemaphoreType.DMA,
            pltpu.SemaphoreType.DMA,
        ],
        compiler_params=pltpu.CompilerParams(use_tc_tiling_on_sc=False),
    )(_emb_body)
    out = k(idx, weight)
    return out.reshape(token_ids.shape + (DIM,))


def kernel(token_ids, weight):
    return _embedding_sc(token_ids, weight)


# speedup vs baseline: 2.4582x; 1.3124x over previous
import functools

import jax
import jax.numpy as jnp
from jax import lax
from jax.experimental import pallas as pl
from jax.experimental.pallas import tpu as pltpu
from jax.experimental.pallas import tpu_sc as plsc

NUM_EMB = 1000000
DIM = 64
BATCH = 16384
SEQ = 50
NUM_WORKERS = 32
B_PER_W = BATCH // NUM_WORKERS
HALF = B_PER_W // 2
N_ITEMS = 2 * SEQ


def _emb_body(idx_hbm, table_hbm, out_hbm, idx_v, rows_a, rows_b, outt_a,
              outt_b, sga, sgb, soa, sob):
    wid = lax.axis_index("s") * 2 + lax.axis_index("c")

    pltpu.sync_copy(idx_hbm.at[:, pl.ds(wid * 4, 4)], idx_v)

    def start_g(i, rows, sem):
        s, h = i // 2, i % 2
        for k in range(2):
            pltpu.async_copy(
                table_hbm.at[idx_v.at[s, h * 2 + k]],
                rows.at[pl.ds(k * 128, 128)], sem)

    def drain_g(i, rows, sem):
        s, h = i // 2, i % 2
        for k in range(2):
            pltpu.make_async_copy(
                table_hbm.at[idx_v.at[s, h * 2 + k]],
                rows.at[pl.ds(k * 128, 128)], sem).wait()

    def transpose(rows_ref, outt_ref):
        iota = lax.iota(jnp.int32, 16)
        rowvs = [iota + bg * 16 for bg in range(16)]
        btvs = [jnp.full((16,), bg // 8, jnp.int32) for bg in range(16)]

        def cbody(ch, _):
            for u in range(2):
                c0 = ch * 2 + u
                colv = jnp.bitwise_and(c0 + iota, 63)
                c8v = lax.shift_right_logical(colv, 3)
                dstb = jnp.bitwise_and(colv, 7) * 128 + iota
                for bg in range(16):
                    val = plsc.load_gather(rows_ref, [rowvs[bg], colv])
                    dstv = dstb + (bg % 8) * 16
                    plsc.store_scatter(outt_ref, [c8v, btvs[bg], dstv], val)
            return 0

        lax.fori_loop(0, DIM // 2, cbody, 0)

    def start_out(i, outt_ref, sem):
        s, h = i // 2, i % 2
        for c8 in range(8):
            pltpu.async_copy(
                outt_ref.at[c8],
                out_hbm.at[s, c8, pl.ds(wid * 4 + h * 2, 2)], sem)

    def wait_out(i, outt_ref, sem):
        s, h = i // 2, i % 2
        for c8 in range(8):
            pltpu.make_async_copy(
                outt_ref.at[c8],
                out_hbm.at[s, c8, pl.ds(wid * 4 + h * 2, 2)], sem).wait()

    start_g(0, rows_a, sga)
    start_g(1, rows_b, sgb)
    drain_g(0, rows_a, sga)
    transpose(rows_a, outt_a)
    start_out(0, outt_a, soa)
    start_g(2, rows_a, sga)
    drain_g(1, rows_b, sgb)
    transpose(rows_b, outt_b)
    start_out(1, outt_b, sob)
    start_g(3, rows_b, sgb)

    def body(j, _):
        i0 = 2 * j + 2
        i1 = 2 * j + 3
        drain_g(i0, rows_a, sga)
        wait_out(i0 - 2, outt_a, soa)
        transpose(rows_a, outt_a)
        start_out(i0, outt_a, soa)
        start_g(i0 + 2, rows_a, sga)
        drain_g(i1, rows_b, sgb)
        wait_out(i1 - 2, outt_b, sob)
        transpose(rows_b, outt_b)
        start_out(i1, outt_b, sob)
        start_g(i1 + 2, rows_b, sgb)
        return 0

    lax.fori_loop(0, (N_ITEMS - 4) // 2, body, 0)

    drain_g(N_ITEMS - 2, rows_a, sga)
    wait_out(N_ITEMS - 4, outt_a, soa)
    transpose(rows_a, outt_a)
    start_out(N_ITEMS - 2, outt_a, soa)
    drain_g(N_ITEMS - 1, rows_b, sgb)
    wait_out(N_ITEMS - 3, outt_b, sob)
    transpose(rows_b, outt_b)
    start_out(N_ITEMS - 1, outt_b, sob)
    wait_out(N_ITEMS - 2, outt_a, soa)
    wait_out(N_ITEMS - 1, outt_b, sob)


@jax.jit
def _embedding_sc(token_ids, weight):
    table = jnp.pad(weight, ((0, 0), (0, DIM))).reshape(2 * NUM_EMB, DIM)
    idx = (token_ids * 2).T.reshape(SEQ, BATCH // 128, 128)
    mesh = plsc.VectorSubcoreMesh(core_axis_name="c", subcore_axis_name="s")
    k = functools.partial(
        pl.kernel,
        mesh=mesh,
        out_type=jax.ShapeDtypeStruct((SEQ, 8, BATCH // 128, 1024),
                                      jnp.float32),
        scratch_types=[
            pltpu.VMEM((SEQ, 4, 128), jnp.int32),
            pltpu.VMEM((HALF, DIM), jnp.float32),
            pltpu.VMEM((HALF, DIM), jnp.float32),
            pltpu.VMEM((8, 2, 1024), jnp.float32),
            pltpu.VMEM((8, 2, 1024), jnp.float32),
            pltpu.SemaphoreType.DMA,
            pltpu.SemaphoreType.DMA,
            pltpu.SemaphoreType.DMA,
            pltpu.SemaphoreType.DMA,
        ],
        compiler_params=pltpu.CompilerParams(use_tc_tiling_on_sc=False,
                                             needs_layout_passes=False),
    )(_emb_body)
    out5 = k(idx, table)
    out5 = out5.reshape(SEQ, 8, BATCH // 128, 8, 128)
    return out5.transpose(2, 4, 0, 1, 3).reshape(BATCH, SEQ, DIM)


def kernel(token_ids, weight):
    return _embedding_sc(token_ids, weight)
